# baseline (speedup 1.0000x reference)
import functools

import jax
import jax.numpy as jnp
from jax import lax
from jax.experimental import pallas as pl
from jax.experimental.pallas import tpu as pltpu
from jax.experimental.pallas import tpu_sc as plsc

N = 50000
E = 800000
B = 500
DIN = 90
H = 128
K = 70

NP = 50176
EP = 802816
EPW = EP // 16
NBLK = EPW // 128
NPT = NP // 16
NZB = 392
CHW = 16
NCH = H // CHW


SBR = 8
NSB = NBLK // SBR


def _conv_sc_body(src2, dst2, w2, xc_hbm, out_hbm,
                  sbig, dbig, wbig, ibig, rows2, zrow, obuf, acc, sem):
    c = lax.axis_index("c")
    s = lax.axis_index("s")
    zi16 = jnp.zeros((16,), jnp.int32)

    def zr(i, carry):
        zrow[i, pl.ds(0, 16)] = jnp.zeros((16,), jnp.float32)
        return carry
    lax.fori_loop(0, NZB, zr, 0)

    for p in range(NCH // 2):
        ch = (NCH // 2) * c + p
        base_off = ch * NP

        def zc(j, carry):
            pltpu.sync_copy(zrow, acc.at[pl.ds(s * NPT + j * NZB, NZB)])
            return carry
        lax.fori_loop(0, 8, zc, 0)
        plsc.subcore_barrier()

        def sb(ib, carry):
            r0 = s * (EPW // 128) + ib * SBR
            pltpu.sync_copy(src2.at[pl.ds(r0, SBR)], sbig)
            pltpu.sync_copy(dst2.at[pl.ds(r0, SBR)], dbig)
            pltpu.sync_copy(w2.at[pl.ds(r0, SBR)], wbig)
            for r in range(SBR):
                for g16 in range(8):
                    ibig[r, pl.ds(g16 * 16, 16)] = (
                        sbig[r, pl.ds(g16 * 16, 16)] + base_off)

            h = pltpu.async_copy(xc_hbm.at[ibig.at[0]], rows2.at[0], sem)
            for b in range(SBR):
                if b < SBR - 1:
                    h_next = pltpu.async_copy(
                        xc_hbm.at[ibig.at[b + 1]], rows2.at[(b + 1) % 2], sem)
                h.wait()
                rb = rows2.at[b % 2]

                def sc(q, cy):
                    wvec = wbig[b, pl.ds(q * 16, 16)]
                    for l in range(16):
                        e = q * 16 + l
                        wsp = wvec.at[zi16 + l].get(mode="promise_in_bounds")
                        rb[e, pl.ds(0, 16)] = rb[e, pl.ds(0, 16)] * wsp
                    return cy
                lax.fori_loop(0, 8, sc, 0)
                pltpu.sync_copy(rb, acc.at[dbig.at[b]], add=True)
                if b < SBR - 1:
                    h = h_next
            return carry
        lax.fori_loop(0, NSB, sb, 0)
        plsc.subcore_barrier()

        def wo(j, carry):
            r0 = s * NPT + j * NZB
            pltpu.sync_copy(acc.at[pl.ds(r0, NZB)], obuf)
            pltpu.sync_copy(obuf, out_hbm.at[pl.ds(base_off + r0, NZB)])
            return carry
        lax.fori_loop(0, 8, wo, 0)
        plsc.subcore_barrier()


_conv_sc = pl.kernel(
    _conv_sc_body,
    out_type=jax.ShapeDtypeStruct((NCH * NP, CHW), jnp.float32),
    mesh=plsc.VectorSubcoreMesh(core_axis_name="c", subcore_axis_name="s"),
    scratch_types=[
        pltpu.VMEM((SBR, 128), jnp.int32),
        pltpu.VMEM((SBR, 128), jnp.int32),
        pltpu.VMEM((SBR, 128), jnp.float32),
        pltpu.VMEM((SBR, 128), jnp.int32),
        pltpu.VMEM((2, 128, CHW), jnp.float32),
        pltpu.VMEM((NZB, CHW), jnp.float32),
        pltpu.VMEM((NZB, CHW), jnp.float32),
        pltpu.VMEM_SHARED((NP, CHW), jnp.float32),
        pltpu.SemaphoreType.DMA,
    ],
    compiler_params=pltpu.CompilerParams(
        use_tc_tiling_on_sc=False, needs_layout_passes=False),
)


DEGR = (EP // 128) // 32


def _deg_sc_body(dst2, w2, out_hbm, dbig, wbig, w16, zrow16, obuf16, acc, sem):
    c = lax.axis_index("c")
    s = lax.axis_index("s")
    lane = lax.broadcasted_iota(jnp.int32, (16,), 0)
    zi16 = jnp.zeros((16,), jnp.int32)
    zf16 = jnp.zeros((16,), jnp.float32)

    def zr(i, carry):
        zrow16[i, pl.ds(0, 16)] = zf16
        return carry

    def zr2(i, carry):
        w16[i, pl.ds(0, 16)] = zf16
        return carry

    def zc(j, carry):
        pltpu.sync_copy(zrow16, acc.at[pl.ds(s * NPT + j * NZB, NZB)])
        return carry

    lax.fori_loop(0, NZB, zr, 0)
    lax.fori_loop(0, 128, zr2, 0)
    lax.fori_loop(0, 8, zc, 0)
    plsc.subcore_barrier()

    def blk(i, carry):
        r0 = (c * 16 + s) * DEGR + i
        pltpu.sync_copy(dst2.at[pl.ds(r0, 1)], dbig)
        pltpu.sync_copy(w2.at[pl.ds(r0, 1)], wbig)
        for q in range(8):
            wv = wbig[0, pl.ds(q * 16, 16)]
            plsc.store_scatter(w16, [lane + q * 16, zi16], wv)
        pltpu.sync_copy(w16, acc.at[dbig.at[0]], add=True)
        return carry
    lax.fori_loop(0, DEGR, blk, 0)
    plsc.subcore_barrier()

    def wo(j, carry):
        r0 = s * NPT + j * NZB
        pltpu.sync_copy(acc.at[pl.ds(r0, NZB)], obuf16)
        pltpu.sync_copy(obuf16, out_hbm.at[pl.ds(c * NP + r0, NZB)])
        return carry
    lax.fori_loop(0, 8, wo, 0)


_deg_sc = pl.kernel(
    _deg_sc_body,
    out_type=jax.ShapeDtypeStruct((2 * NP, 16), jnp.float32),
    mesh=plsc.VectorSubcoreMesh(core_axis_name="c", subcore_axis_name="s"),
    scratch_types=[
        pltpu.VMEM((1, 128), jnp.int32),
        pltpu.VMEM((1, 128), jnp.float32),
        pltpu.VMEM((128, 16), jnp.float32),
        pltpu.VMEM((NZB, 16), jnp.float32),
        pltpu.VMEM((NZB, 16), jnp.float32),
        pltpu.VMEM_SHARED((NP, 16), jnp.float32),
        pltpu.SemaphoreType.DMA,
    ],
    compiler_params=pltpu.CompilerParams(
        use_tc_tiling_on_sc=False, needs_layout_passes=False),
)


GPW = 16
DNP = 35848
DUMP = 35840
RB = 64


def _pool_body(v_hbm, h2_hbm, batch_hbm, out_hbm,
               vbuf, bbuf, zbuf, rowbuf, idxbuf, sem):
    c = lax.axis_index("c")
    s = lax.axis_index("s")
    w = s * 2 + c
    pltpu.sync_copy(v_hbm, vbuf)
    pltpu.sync_copy(batch_hbm, bbuf)

    def zr(i, cy):
        for hh in range(8):
            zbuf[i, pl.ds(hh * 16, 16)] = jnp.zeros((16,), jnp.float32)
        return cy
    lax.fori_loop(0, 56, zr, 0)

    r0 = w * (GPW * K)

    def zo(kk, cy):
        pltpu.sync_copy(zbuf, out_hbm.at[pl.ds(r0 + kk * 56, 56)])
        return cy
    lax.fori_loop(0, 20, zo, 0)

    lane = lax.broadcasted_iota(jnp.int32, (16,), 0)
    zi16 = jnp.zeros((16,), jnp.int32)

    def search(gv):
        def it(_, carry):
            lo, hi = carry
            mid = (lo + hi) // 2
            bv = plsc.load_gather(bbuf, [mid])
            pred = bv < gv
            return (jnp.where(pred, mid + 1, lo), jnp.where(pred, hi, mid))
        lo, _ = lax.fori_loop(0, 17, it, (zi16, zi16 + N))
        return lo

    gv = w * GPW + lane
    sv = search(gv)
    ev = search(gv + 1)

    def graph_body(j, cy0):
        g = w * GPW + j
        sg = sv.at[zi16 + j].get(mode="promise_in_bounds")[0]
        eg = ev.at[zi16 + j].get(mode="promise_in_bounds")[0]
        n = eg - sg
        nb = (n + RB - 1) // RB

        def blk(kk, cy):
            b0 = sg + kk * RB
            pltpu.sync_copy(h2_hbm.at[pl.ds(b0, RB)], rowbuf)
            njc = (n + 15) // 16
            for q in range(RB // 16):
                loc = kk * RB + q * 16 + lane
                gi = sg + loc
                viv = plsc.load_gather(vbuf, [gi])
                valid = loc < n

                def jl(jb, rank):
                    jbase = sg + jb * 16
                    vjv = plsc.load_gather(vbuf, [jbase + lane])
                    for l in range(16):
                        jg = jbase + l
                        vjs = vjv.at[zi16 + l].get(mode="promise_in_bounds")
                        win = ((vjs > viv)
                               | ((vjs == viv) & (jg < gi))) & (jg < eg)
                        rank = rank + jnp.where(win, 1, 0)
                    return rank
                rank = lax.fori_loop(0, njc, jl, zi16)
                outi = jnp.where(valid & (rank < K), g * K + rank, DUMP)
                idxbuf[pl.ds(q * 16, 16)] = outi
            pltpu.sync_copy(rowbuf, out_hbm.at[idxbuf])
            return cy
        lax.fori_loop(0, nb, blk, 0)
        return cy0
    lax.fori_loop(0, GPW, graph_body, 0)


_pool_sc = pl.kernel(
    _pool_body,
    out_type=jax.ShapeDtypeStruct((DNP, H), jnp.float32),
    mesh=plsc.VectorSubcoreMesh(core_axis_name="c", subcore_axis_name="s"),
    scratch_types=[
        pltpu.VMEM((NP,), jnp.float32),
        pltpu.VMEM((NP,), jnp.int32),
        pltpu.VMEM((56, H), jnp.float32),
        pltpu.VMEM((RB, H), jnp.float32),
        pltpu.VMEM((RB,), jnp.int32),
        pltpu.SemaphoreType.DMA,
    ],
    compiler_params=pltpu.CompilerParams(
        use_tc_tiling_on_sc=False, needs_layout_passes=False),
)


DPAD = 96


def _to_chunks(xp):
    return jnp.transpose(xp.reshape(NP, NCH, CHW), (1, 0, 2)).reshape(
        NCH * NP, CHW)


def _from_chunks(agg):
    return jnp.transpose(agg.reshape(NCH, NP, CHW), (1, 0, 2)).reshape(NP, H)


def _k2_body(x_ref, w1_ref, degp_ref, x1_ref, dinv_ref):
    deg = 1.0 + degp_ref[0][:, 0:1] + degp_ref[1][:, 0:1]
    dinv = lax.rsqrt(deg)
    x1_ref[...] = jnp.dot(x_ref[...], w1_ref[...],
                          preferred_element_type=jnp.float32) * dinv
    dinv_ref[...] = dinv


def _k2(x_pad, W1p, degp):
    MB = 512
    return pl.pallas_call(
        _k2_body,
        grid=(NP // MB,),
        in_specs=[
            pl.BlockSpec((MB, DPAD), lambda i: (i, 0)),
            pl.BlockSpec((DPAD, H), lambda i: (0, 0)),
            pl.BlockSpec((2, MB, 16), lambda i: (0, i, 0)),
        ],
        out_specs=[
            pl.BlockSpec((MB, H), lambda i: (i, 0)),
            pl.BlockSpec((MB, 1), lambda i: (i, 0)),
        ],
        out_shape=[
            jax.ShapeDtypeStruct((NP, H), jnp.float32),
            jax.ShapeDtypeStruct((NP, 1), jnp.float32),
        ],
    )(x_pad, W1p, degp)


def _k4_body(agg_ref, x1_ref, dinv_ref, b_ref, w2_ref, xt_ref, x2_ref):
    dinv = dinv_ref[...]
    h1 = dinv * (agg_ref[...] + x1_ref[...]) + b_ref[...]
    xt_ref[...] = h1
    x2_ref[...] = jnp.dot(jnp.maximum(h1, 0.0), w2_ref[...],
                          preferred_element_type=jnp.float32) * dinv


def _k4(agg1, x1, dinv, b1, W2):
    MB = 512
    return pl.pallas_call(
        _k4_body,
        grid=(NP // MB,),
        in_specs=[
            pl.BlockSpec((MB, H), lambda i: (i, 0)),
            pl.BlockSpec((MB, H), lambda i: (i, 0)),
            pl.BlockSpec((MB, 1), lambda i: (i, 0)),
            pl.BlockSpec((1, H), lambda i: (0, 0)),
            pl.BlockSpec((H, H), lambda i: (0, 0)),
        ],
        out_specs=[
            pl.BlockSpec((MB, H), lambda i: (i, 0)),
            pl.BlockSpec((MB, H), lambda i: (i, 0)),
        ],
        out_shape=[
            jax.ShapeDtypeStruct((NP, H), jnp.float32),
            jax.ShapeDtypeStruct((NP, H), jnp.float32),
        ],
    )(agg1, x1, dinv, b1, W2)


def _k6_body(agg_ref, x2_ref, dinv_ref, b_ref, h2_ref, v_ref):
    h2 = dinv_ref[...] * (agg_ref[...] + x2_ref[...]) + b_ref[...]
    h2_ref[...] = h2
    v_ref[...] = h2[:, H - 1:H]


def _k6(agg2, x2, dinv, b2):
    MB = 512
    return pl.pallas_call(
        _k6_body,
        grid=(NP // MB,),
        in_specs=[
            pl.BlockSpec((MB, H), lambda i: (i, 0)),
            pl.BlockSpec((MB, H), lambda i: (i, 0)),
            pl.BlockSpec((MB, 1), lambda i: (i, 0)),
            pl.BlockSpec((1, H), lambda i: (0, 0)),
        ],
        out_specs=[
            pl.BlockSpec((MB, H), lambda i: (i, 0)),
            pl.BlockSpec((MB, 1), lambda i: (i, 0)),
        ],
        out_shape=[
            jax.ShapeDtypeStruct((NP, H), jnp.float32),
            jax.ShapeDtypeStruct((NP, 1), jnp.float32),
        ],
    )(agg2, x2, dinv, b2)


def _head_body(hp_ref, wl1_ref, bl1_ref, wl2_ref, bl2_ref, o_ref, acc):
    j = pl.program_id(1)
    nk = pl.num_programs(1)

    @pl.when(j == 0)
    def _():
        acc[...] = jnp.zeros_like(acc)

    acc[...] += jnp.dot(hp_ref[...], wl1_ref[...],
                        preferred_element_type=jnp.float32)

    @pl.when(j == nk - 1)
    def _():
        t = acc[...] + bl1_ref[...]
        z = jnp.dot(t, wl2_ref[...], preferred_element_type=jnp.float32)
        o_ref[...] = jax.nn.sigmoid(z + bl2_ref[...])


def _dense_head(hp, Wl1, bl1, Wl2, bl2):
    MB = 128
    KB = 1280
    mpad = 512
    hp = jnp.pad(hp, ((0, mpad - B), (0, 0)))
    out = pl.pallas_call(
        _head_body,
        grid=(mpad // MB, (K * H) // KB),
        in_specs=[
            pl.BlockSpec((MB, KB), lambda i, j: (i, j)),
            pl.BlockSpec((KB, H), lambda i, j: (j, 0)),
            pl.BlockSpec((1, H), lambda i, j: (0, 0)),
            pl.BlockSpec((H, 1), lambda i, j: (0, 0)),
            pl.BlockSpec((1, 1), lambda i, j: (0, 0)),
        ],
        out_specs=pl.BlockSpec((MB, 1), lambda i, j: (i, 0)),
        out_shape=jax.ShapeDtypeStruct((mpad, 1), jnp.float32),
        scratch_shapes=[pltpu.VMEM((MB, H), jnp.float32)],
    )(hp, Wl1, bl1.reshape(1, H), Wl2, bl2.reshape(1, 1))
    return out[:B, 0]


def kernel(x, edge_index, edge_weight, batch, W1, b1, W2, b2, Wl1, bl1, Wl2, bl2):
    src, dst = edge_index[0], edge_index[1]
    src_p = jnp.pad(src, (0, EP - E)).reshape(EP // 128, 128)
    dst_p = jnp.pad(dst, (0, EP - E)).reshape(EP // 128, 128)
    w_p = jnp.pad(edge_weight, (0, EP - E)).reshape(EP // 128, 128)

    degp = _deg_sc(dst_p, w_p).reshape(2, NP, 16)
    x_pad = jnp.pad(x, ((0, NP - N), (0, DPAD - DIN)))
    W1p = jnp.pad(W1, ((0, DPAD - DIN), (0, 0)))
    x1, dinv = _k2(x_pad, W1p, degp)
    agg1 = _conv_sc(src_p, dst_p, w_p, _to_chunks(x1))
    xt_p, x2 = _k4(_from_chunks(agg1), x1, dinv, b1.reshape(1, H), W2)
    x_train = xt_p[:N]
    agg2 = _conv_sc(src_p, dst_p, w_p, _to_chunks(x2))
    h2p, vp = _k6(_from_chunks(agg2), x2, dinv, b2.reshape(1, H))

    batch_p = jnp.pad(batch, (0, NP - N), constant_values=1000)
    dense = _pool_sc(vp.reshape(NP), h2p, batch_p)
    hp = dense[:B * K].reshape(B, K * H)
    out = _dense_head(hp, Wl1, bl1, Wl2, bl2)
    return (out, x_train)


# speedup vs baseline: 1.0752x; 1.0752x over previous
import functools

import jax
import jax.numpy as jnp
from jax import lax
from jax.experimental import pallas as pl
from jax.experimental.pallas import tpu as pltpu
from jax.experimental.pallas import tpu_sc as plsc

N = 50000
E = 800000
B = 500
DIN = 90
H = 128
K = 70

NP = 50176
EP = 802816
EPW = EP // 16
NBLK = EPW // 128
NPT = NP // 16
NZB = 392
CHW = 16
NCH = H // CHW


SBR = 8
NSB = NBLK // SBR


def _conv_sc_body(src2, dst2, w2, xc_hbm, out_hbm,
                  sbig, dbig, wbig, ibig, rows2, zrow, obuf, acc, sem):
    c = lax.axis_index("c")
    s = lax.axis_index("s")
    zi16 = jnp.zeros((16,), jnp.int32)

    def zr(i, carry):
        zrow[i, pl.ds(0, 16)] = jnp.zeros((16,), jnp.float32)
        return carry
    lax.fori_loop(0, NZB, zr, 0)

    for p in range(NCH // 2):
        ch = (NCH // 2) * c + p
        base_off = ch * NP

        def zc(j, carry):
            pltpu.sync_copy(zrow, acc.at[pl.ds(s * NPT + j * NZB, NZB)])
            return carry
        lax.fori_loop(0, 8, zc, 0)
        plsc.subcore_barrier()

        def sb(ib, carry):
            r0 = s * (EPW // 128) + ib * SBR
            pltpu.sync_copy(src2.at[pl.ds(r0, SBR)], sbig)
            pltpu.sync_copy(dst2.at[pl.ds(r0, SBR)], dbig)
            pltpu.sync_copy(w2.at[pl.ds(r0, SBR)], wbig)
            for r in range(SBR):
                for g16 in range(8):
                    ibig[r, pl.ds(g16 * 16, 16)] = (
                        sbig[r, pl.ds(g16 * 16, 16)] + base_off)

            hs = [
                pltpu.async_copy(xc_hbm.at[ibig.at[0]], rows2.at[0], sem),
                pltpu.async_copy(xc_hbm.at[ibig.at[1]], rows2.at[1], sem),
            ]
            for b in range(SBR):
                if b < SBR - 2:
                    hs.append(pltpu.async_copy(
                        xc_hbm.at[ibig.at[b + 2]], rows2.at[(b + 2) % 3], sem))
                hs[b].wait()
                rb = rows2.at[b % 3]

                def sc(q, cy):
                    wvec = wbig[b, pl.ds(q * 16, 16)]
                    for l in range(16):
                        e = q * 16 + l
                        wsp = wvec.at[zi16 + l].get(mode="promise_in_bounds")
                        rb[e, pl.ds(0, 16)] = rb[e, pl.ds(0, 16)] * wsp
                    return cy
                lax.fori_loop(0, 8, sc, 0)
                pltpu.sync_copy(rb, acc.at[dbig.at[b]], add=True)
            return carry
        lax.fori_loop(0, NSB, sb, 0)
        plsc.subcore_barrier()

        def wo(j, carry):
            r0 = s * NPT + j * NZB
            pltpu.sync_copy(acc.at[pl.ds(r0, NZB)], obuf)
            pltpu.sync_copy(obuf, out_hbm.at[pl.ds(base_off + r0, NZB)])
            return carry
        lax.fori_loop(0, 8, wo, 0)
        plsc.subcore_barrier()


_conv_sc = pl.kernel(
    _conv_sc_body,
    out_type=jax.ShapeDtypeStruct((NCH * NP, CHW), jnp.float32),
    mesh=plsc.VectorSubcoreMesh(core_axis_name="c", subcore_axis_name="s"),
    scratch_types=[
        pltpu.VMEM((SBR, 128), jnp.int32),
        pltpu.VMEM((SBR, 128), jnp.int32),
        pltpu.VMEM((SBR, 128), jnp.float32),
        pltpu.VMEM((SBR, 128), jnp.int32),
        pltpu.VMEM((3, 128, CHW), jnp.float32),
        pltpu.VMEM((NZB, CHW), jnp.float32),
        pltpu.VMEM((NZB, CHW), jnp.float32),
        pltpu.VMEM_SHARED((NP, CHW), jnp.float32),
        pltpu.SemaphoreType.DMA,
    ],
    compiler_params=pltpu.CompilerParams(
        use_tc_tiling_on_sc=False, needs_layout_passes=False),
)


DEGR = (EP // 128) // 32


def _deg_sc_body(dst2, w2, out_hbm, dbig, wbig, w16, zrow16, obuf16, acc, sem):
    c = lax.axis_index("c")
    s = lax.axis_index("s")
    lane = lax.broadcasted_iota(jnp.int32, (16,), 0)
    zi16 = jnp.zeros((16,), jnp.int32)
    zf16 = jnp.zeros((16,), jnp.float32)

    def zr(i, carry):
        zrow16[i, pl.ds(0, 16)] = zf16
        return carry

    def zr2(i, carry):
        w16[i, pl.ds(0, 16)] = zf16
        return carry

    def zc(j, carry):
        pltpu.sync_copy(zrow16, acc.at[pl.ds(s * NPT + j * NZB, NZB)])
        return carry

    lax.fori_loop(0, NZB, zr, 0)
    lax.fori_loop(0, 128, zr2, 0)
    lax.fori_loop(0, 8, zc, 0)
    plsc.subcore_barrier()

    def blk(i, carry):
        r0 = (c * 16 + s) * DEGR + i
        pltpu.sync_copy(dst2.at[pl.ds(r0, 1)], dbig)
        pltpu.sync_copy(w2.at[pl.ds(r0, 1)], wbig)
        for q in range(8):
            wv = wbig[0, pl.ds(q * 16, 16)]
            plsc.store_scatter(w16, [lane + q * 16, zi16], wv)
        pltpu.sync_copy(w16, acc.at[dbig.at[0]], add=True)
        return carry
    lax.fori_loop(0, DEGR, blk, 0)
    plsc.subcore_barrier()

    def wo(j, carry):
        r0 = s * NPT + j * NZB
        pltpu.sync_copy(acc.at[pl.ds(r0, NZB)], obuf16)
        pltpu.sync_copy(obuf16, out_hbm.at[pl.ds(c * NP + r0, NZB)])
        return carry
    lax.fori_loop(0, 8, wo, 0)


_deg_sc = pl.kernel(
    _deg_sc_body,
    out_type=jax.ShapeDtypeStruct((2 * NP, 16), jnp.float32),
    mesh=plsc.VectorSubcoreMesh(core_axis_name="c", subcore_axis_name="s"),
    scratch_types=[
        pltpu.VMEM((1, 128), jnp.int32),
        pltpu.VMEM((1, 128), jnp.float32),
        pltpu.VMEM((128, 16), jnp.float32),
        pltpu.VMEM((NZB, 16), jnp.float32),
        pltpu.VMEM((NZB, 16), jnp.float32),
        pltpu.VMEM_SHARED((NP, 16), jnp.float32),
        pltpu.SemaphoreType.DMA,
    ],
    compiler_params=pltpu.CompilerParams(
        use_tc_tiling_on_sc=False, needs_layout_passes=False),
)


GPW = 16
DNP = 35848
DUMP = 35840
RB = 64


def _pool_body(v_hbm, h2_hbm, batch_hbm, out_hbm,
               vbuf, bbuf, zbuf, rowbuf, idxbuf, sem):
    c = lax.axis_index("c")
    s = lax.axis_index("s")
    w = s * 2 + c
    pltpu.sync_copy(v_hbm, vbuf)
    pltpu.sync_copy(batch_hbm, bbuf)

    def zr(i, cy):
        for hh in range(8):
            zbuf[i, pl.ds(hh * 16, 16)] = jnp.zeros((16,), jnp.float32)
        return cy
    lax.fori_loop(0, 56, zr, 0)

    r0 = w * (GPW * K)

    def zo(kk, cy):
        pltpu.sync_copy(zbuf, out_hbm.at[pl.ds(r0 + kk * 56, 56)])
        return cy
    lax.fori_loop(0, 20, zo, 0)

    lane = lax.broadcasted_iota(jnp.int32, (16,), 0)
    zi16 = jnp.zeros((16,), jnp.int32)

    def search(gv):
        def it(_, carry):
            lo, hi = carry
            mid = (lo + hi) // 2
            bv = plsc.load_gather(bbuf, [mid])
            pred = bv < gv
            return (jnp.where(pred, mid + 1, lo), jnp.where(pred, hi, mid))
        lo, _ = lax.fori_loop(0, 17, it, (zi16, zi16 + N))
        return lo

    gv = w * GPW + lane
    sv = search(gv)
    ev = search(gv + 1)

    def graph_body(j, cy0):
        g = w * GPW + j
        sg = sv.at[zi16 + j].get(mode="promise_in_bounds")[0]
        eg = ev.at[zi16 + j].get(mode="promise_in_bounds")[0]
        n = eg - sg
        nb = (n + RB - 1) // RB

        def blk(kk, cy):
            b0 = sg + kk * RB
            pltpu.sync_copy(h2_hbm.at[pl.ds(b0, RB)], rowbuf)
            njc = (n + 15) // 16
            for q in range(RB // 16):
                loc = kk * RB + q * 16 + lane
                gi = sg + loc
                viv = plsc.load_gather(vbuf, [gi])
                valid = loc < n

                def jl(jb, rank):
                    jbase = sg + jb * 16
                    vjv = plsc.load_gather(vbuf, [jbase + lane])
                    for l in range(16):
                        jg = jbase + l
                        vjs = vjv.at[zi16 + l].get(mode="promise_in_bounds")
                        win = ((vjs > viv)
                               | ((vjs == viv) & (jg < gi))) & (jg < eg)
                        rank = rank + jnp.where(win, 1, 0)
                    return rank
                rank = lax.fori_loop(0, njc, jl, zi16)
                outi = jnp.where(valid & (rank < K), g * K + rank, DUMP)
                idxbuf[pl.ds(q * 16, 16)] = outi
            pltpu.sync_copy(rowbuf, out_hbm.at[idxbuf])
            return cy
        lax.fori_loop(0, nb, blk, 0)
        return cy0
    lax.fori_loop(0, GPW, graph_body, 0)


_pool_sc = pl.kernel(
    _pool_body,
    out_type=jax.ShapeDtypeStruct((DNP, H), jnp.float32),
    mesh=plsc.VectorSubcoreMesh(core_axis_name="c", subcore_axis_name="s"),
    scratch_types=[
        pltpu.VMEM((NP,), jnp.float32),
        pltpu.VMEM((NP,), jnp.int32),
        pltpu.VMEM((56, H), jnp.float32),
        pltpu.VMEM((RB, H), jnp.float32),
        pltpu.VMEM((RB,), jnp.int32),
        pltpu.SemaphoreType.DMA,
    ],
    compiler_params=pltpu.CompilerParams(
        use_tc_tiling_on_sc=False, needs_layout_passes=False),
)


DPAD = 96


def _to_chunks(xp):
    return jnp.transpose(xp.reshape(NP, NCH, CHW), (1, 0, 2)).reshape(
        NCH * NP, CHW)


def _from_chunks(agg):
    return jnp.transpose(agg.reshape(NCH, NP, CHW), (1, 0, 2)).reshape(NP, H)


def _k2_body(x_ref, w1_ref, degp_ref, x1_ref, dinv_ref):
    deg = 1.0 + degp_ref[0][:, 0:1] + degp_ref[1][:, 0:1]
    dinv = lax.rsqrt(deg)
    x1_ref[...] = jnp.dot(x_ref[...], w1_ref[...],
                          preferred_element_type=jnp.float32) * dinv
    dinv_ref[...] = dinv


def _k2(x_pad, W1p, degp):
    MB = 512
    return pl.pallas_call(
        _k2_body,
        grid=(NP // MB,),
        in_specs=[
            pl.BlockSpec((MB, DPAD), lambda i: (i, 0)),
            pl.BlockSpec((DPAD, H), lambda i: (0, 0)),
            pl.BlockSpec((2, MB, 16), lambda i: (0, i, 0)),
        ],
        out_specs=[
            pl.BlockSpec((MB, H), lambda i: (i, 0)),
            pl.BlockSpec((MB, 1), lambda i: (i, 0)),
        ],
        out_shape=[
            jax.ShapeDtypeStruct((NP, H), jnp.float32),
            jax.ShapeDtypeStruct((NP, 1), jnp.float32),
        ],
    )(x_pad, W1p, degp)


def _k4_body(agg_ref, x1_ref, dinv_ref, b_ref, w2_ref, xt_ref, x2_ref):
    dinv = dinv_ref[...]
    h1 = dinv * (agg_ref[...] + x1_ref[...]) + b_ref[...]
    xt_ref[...] = h1
    x2_ref[...] = jnp.dot(jnp.maximum(h1, 0.0), w2_ref[...],
                          preferred_element_type=jnp.float32) * dinv


def _k4(agg1, x1, dinv, b1, W2):
    MB = 512
    return pl.pallas_call(
        _k4_body,
        grid=(NP // MB,),
        in_specs=[
            pl.BlockSpec((MB, H), lambda i: (i, 0)),
            pl.BlockSpec((MB, H), lambda i: (i, 0)),
            pl.BlockSpec((MB, 1), lambda i: (i, 0)),
            pl.BlockSpec((1, H), lambda i: (0, 0)),
            pl.BlockSpec((H, H), lambda i: (0, 0)),
        ],
        out_specs=[
            pl.BlockSpec((MB, H), lambda i: (i, 0)),
            pl.BlockSpec((MB, H), lambda i: (i, 0)),
        ],
        out_shape=[
            jax.ShapeDtypeStruct((NP, H), jnp.float32),
            jax.ShapeDtypeStruct((NP, H), jnp.float32),
        ],
    )(agg1, x1, dinv, b1, W2)


def _k6_body(agg_ref, x2_ref, dinv_ref, b_ref, h2_ref, v_ref):
    h2 = dinv_ref[...] * (agg_ref[...] + x2_ref[...]) + b_ref[...]
    h2_ref[...] = h2
    v_ref[...] = h2[:, H - 1:H]


def _k6(agg2, x2, dinv, b2):
    MB = 512
    return pl.pallas_call(
        _k6_body,
        grid=(NP // MB,),
        in_specs=[
            pl.BlockSpec((MB, H), lambda i: (i, 0)),
            pl.BlockSpec((MB, H), lambda i: (i, 0)),
            pl.BlockSpec((MB, 1), lambda i: (i, 0)),
            pl.BlockSpec((1, H), lambda i: (0, 0)),
        ],
        out_specs=[
            pl.BlockSpec((MB, H), lambda i: (i, 0)),
            pl.BlockSpec((MB, 1), lambda i: (i, 0)),
        ],
        out_shape=[
            jax.ShapeDtypeStruct((NP, H), jnp.float32),
            jax.ShapeDtypeStruct((NP, 1), jnp.float32),
        ],
    )(agg2, x2, dinv, b2)


def _head_body(hp_ref, wl1_ref, bl1_ref, wl2_ref, bl2_ref, o_ref, acc):
    j = pl.program_id(1)
    nk = pl.num_programs(1)

    @pl.when(j == 0)
    def _():
        acc[...] = jnp.zeros_like(acc)

    acc[...] += jnp.dot(hp_ref[...], wl1_ref[...],
                        preferred_element_type=jnp.float32)

    @pl.when(j == nk - 1)
    def _():
        t = acc[...] + bl1_ref[...]
        z = jnp.dot(t, wl2_ref[...], preferred_element_type=jnp.float32)
        o_ref[...] = jax.nn.sigmoid(z + bl2_ref[...])


def _dense_head(hp, Wl1, bl1, Wl2, bl2):
    MB = 128
    KB = 1280
    mpad = 512
    hp = jnp.pad(hp, ((0, mpad - B), (0, 0)))
    out = pl.pallas_call(
        _head_body,
        grid=(mpad // MB, (K * H) // KB),
        in_specs=[
            pl.BlockSpec((MB, KB), lambda i, j: (i, j)),
            pl.BlockSpec((KB, H), lambda i, j: (j, 0)),
            pl.BlockSpec((1, H), lambda i, j: (0, 0)),
            pl.BlockSpec((H, 1), lambda i, j: (0, 0)),
            pl.BlockSpec((1, 1), lambda i, j: (0, 0)),
        ],
        out_specs=pl.BlockSpec((MB, 1), lambda i, j: (i, 0)),
        out_shape=jax.ShapeDtypeStruct((mpad, 1), jnp.float32),
        scratch_shapes=[pltpu.VMEM((MB, H), jnp.float32)],
    )(hp, Wl1, bl1.reshape(1, H), Wl2, bl2.reshape(1, 1))
    return out[:B, 0]


def kernel(x, edge_index, edge_weight, batch, W1, b1, W2, b2, Wl1, bl1, Wl2, bl2):
    src, dst = edge_index[0], edge_index[1]
    src_p = jnp.pad(src, (0, EP - E)).reshape(EP // 128, 128)
    dst_p = jnp.pad(dst, (0, EP - E)).reshape(EP // 128, 128)
    w_p = jnp.pad(edge_weight, (0, EP - E)).reshape(EP // 128, 128)

    degp = _deg_sc(dst_p, w_p).reshape(2, NP, 16)
    x_pad = jnp.pad(x, ((0, NP - N), (0, DPAD - DIN)))
    W1p = jnp.pad(W1, ((0, DPAD - DIN), (0, 0)))
    x1, dinv = _k2(x_pad, W1p, degp)
    agg1 = _conv_sc(src_p, dst_p, w_p, _to_chunks(x1))
    xt_p, x2 = _k4(_from_chunks(agg1), x1, dinv, b1.reshape(1, H), W2)
    x_train = xt_p[:N]
    agg2 = _conv_sc(src_p, dst_p, w_p, _to_chunks(x2))
    h2p, vp = _k6(_from_chunks(agg2), x2, dinv, b2.reshape(1, H))

    batch_p = jnp.pad(batch, (0, NP - N), constant_values=1000)
    dense = _pool_sc(vp.reshape(NP), h2p, batch_p)
    hp = dense[:B * K].reshape(B, K * H)
    out = _dense_head(hp, Wl1, bl1, Wl2, bl2)
    return (out, x_train)
